# baseline (speedup 1.0000x reference)
import functools

import numpy as np
import jax
import jax.numpy as jnp
from jax import lax
from jax.experimental import pallas as pl
from jax.experimental.pallas import tpu as pltpu
from jax.experimental.pallas import tpu_sc as plsc

N = 10000
D = 128
H = 64
NB = 105
NS = 105
E = 320000

NC = 2
NSC = 16
NW = NC * NSC
NPAD = 10240
SLAB = NPAD // NSC
CHUNK = 128
CPT = 80
CPT0 = 160
CPT1 = 0
CH0 = NSC * CPT0
TOT_CH = NSC * (CPT0 + CPT1)
ALLOC_CH = CH0 + (NSC - 1) * CPT1 + CPT0
E_PAD = ALLOC_CH * CHUNK
NBUF = 2
DW = 16

BR = 1024
GRID = NPAD // BR

@functools.lru_cache
def _mesh():
    return plsc.VectorSubcoreMesh(core_axis_name="c", subcore_axis_name="s",
                                  num_cores=NC, num_subcores=NSC)


_GCACHE = []


def _gumbels():
    if _GCACHE:
        return _GCACHE[0]

    def build():
        gk1, gk2 = jax.random.split(jax.random.key(42))
        outs = []
        for k, n in ((gk1, NB), (gk2, NS)):
            u = jax.random.uniform(k, (N, n), minval=1e-6, maxval=1.0 - 1e-6)
            outs.append(jnp.pad(-jnp.log(-jnp.log(u)), ((0, NPAD - N), (0, 0))))
        return outs

    try:
        with jax.ensure_compile_time_eval():
            gb, gs = build()
        gb, gs = np.asarray(gb), np.asarray(gs)
        _GCACHE.append((gb, gs))
        return gb, gs
    except Exception:
        return build()


def _deg_stem_body(dst_hbm, stem_hbm, ones_hbm, z1_hbm, degp_out, stem_out,
                   dstv, stemv, onesv, deg_sp, stem_sp, sem, sem2):
    cid = lax.axis_index("c")
    sid = lax.axis_index("s")
    wid = sid * NC + cid
    row0 = sid * SLAB
    pltpu.sync_copy(z1_hbm, deg_sp.at[pl.ds(row0, SLAB)])

    @pl.when(cid == 0)
    def _():
        pltpu.sync_copy(z1_hbm, stem_sp.at[pl.ds(row0, SLAB)])

    pltpu.sync_copy(ones_hbm, onesv)
    pltpu.sync_copy(dst_hbm.at[pl.ds(wid * CPT, CPT)], dstv)
    is00 = jnp.logical_and(cid == 0, sid == 0)

    @pl.when(is00)
    def _():
        pltpu.sync_copy(stem_hbm, stemv)

    plsc.subcore_barrier()

    def body(i, carry):
        for b in range(4):
            pltpu.async_copy(onesv, deg_sp.at[dstv.at[i * 4 + b]], sem, add=True)
        for _ in range(4):
            pltpu.make_async_copy(onesv, deg_sp.at[dstv.at[0]], sem).wait()
        return carry

    lax.fori_loop(0, CPT // 4, body, 0)

    @pl.when(is00)
    def _():
        for j in range(4):
            pltpu.async_copy(onesv, stem_sp.at[stemv.at[j]], sem2)
        for _ in range(4):
            pltpu.make_async_copy(onesv, stem_sp.at[stemv.at[0]], sem2).wait()

    plsc.subcore_barrier()
    pltpu.sync_copy(deg_sp.at[pl.ds(row0, SLAB)],
                    degp_out.at[cid, pl.ds(row0, SLAB)])

    @pl.when(cid == 0)
    def _():
        pltpu.sync_copy(stem_sp.at[pl.ds(row0, SLAB)],
                        stem_out.at[pl.ds(row0, SLAB)])


def _agg_body(src_hbm, dst_hbm, y_hbm, z64_hbm, out,
              srcv, dstv, agg_sp, buf0, buf1,
              g0, g1, s0, s1):
    bufs = (buf0, buf1)
    gs = (g0, g1)
    ss = (s0, s1)
    cid = lax.axis_index("c")
    sid = lax.axis_index("s")
    row0 = sid * SLAB
    base = jnp.where(cid == 0, sid * CPT0, CH0 + sid * CPT1)
    pltpu.sync_copy(z64_hbm, agg_sp.at[pl.ds(row0, SLAB)])
    pltpu.sync_copy(src_hbm.at[pl.ds(base, CPT0)], srcv)
    pltpu.sync_copy(dst_hbm.at[pl.ds(base, CPT0)], dstv)
    plsc.subcore_barrier()

    def run(my_cpt):
        if my_cpt == 0:
            return
        for b in range(NBUF):
            pltpu.async_copy(y_hbm.at[srcv.at[b]], bufs[b], gs[b])

        def body(i, carry):
            for b in range(NBUF):
                c = i * NBUF + b
                pltpu.make_async_copy(y_hbm.at[srcv.at[0]], bufs[b],
                                      gs[b]).wait()
                pltpu.async_copy(bufs[b], agg_sp.at[dstv.at[c]], ss[b],
                                 add=True)
                pltpu.make_async_copy(bufs[b], agg_sp.at[dstv.at[0]],
                                      ss[b]).wait()
                nc = c + NBUF

                @pl.when(nc < my_cpt)
                def _():
                    pltpu.async_copy(y_hbm.at[srcv.at[nc]], bufs[b], gs[b])
            return carry

        lax.fori_loop(0, my_cpt // NBUF, body, 0)

    @pl.when(cid == 0)
    def _():
        run(CPT0)

    @pl.when(cid == 1)
    def _():
        run(CPT1)

    plsc.subcore_barrier()
    pltpu.sync_copy(agg_sp.at[pl.ds(row0, SLAB)],
                    out.at[cid, pl.ds(row0, SLAB)])


@functools.lru_cache
def _deg_stem_kernel():
    return pl.kernel(
        _deg_stem_body,
        out_type=(jax.ShapeDtypeStruct((NC, NPAD, DW), jnp.float32),
                  jax.ShapeDtypeStruct((NPAD, DW), jnp.float32)),
        mesh=_mesh(),
        scratch_types=[
            pltpu.VMEM((CPT, CHUNK), jnp.int32),
            pltpu.VMEM((4, CHUNK), jnp.int32),
            pltpu.VMEM((CHUNK, DW), jnp.float32),
            pltpu.VMEM_SHARED((NPAD, DW), jnp.float32),
            pltpu.VMEM_SHARED((NPAD, DW), jnp.float32),
            pltpu.SemaphoreType.DMA,
            pltpu.SemaphoreType.DMA,
        ],
        compiler_params=pltpu.CompilerParams(use_tc_tiling_on_sc=False),
    )


def _deg_stem(dstp, stemp, ones_col, z1):
    return _deg_stem_kernel()(dstp, stemp, ones_col, z1)


@functools.lru_cache
def _agg_kernel():
    return pl.kernel(
        _agg_body,
        out_type=jax.ShapeDtypeStruct((NC, NPAD, H), jnp.float32),
        mesh=_mesh(),
        scratch_types=[
            pltpu.VMEM((CPT0, CHUNK), jnp.int32),
            pltpu.VMEM((CPT0, CHUNK), jnp.int32),
            pltpu.VMEM_SHARED((NPAD, H), jnp.float32),
            pltpu.VMEM((CHUNK, H), jnp.float32),
            pltpu.VMEM((CHUNK, H), jnp.float32),
            pltpu.SemaphoreType.DMA,
            pltpu.SemaphoreType.DMA,
            pltpu.SemaphoreType.DMA,
            pltpu.SemaphoreType.DMA,
        ],
        compiler_params=pltpu.CompilerParams(use_tc_tiling_on_sc=False),
    )


def _agg(srcp, dstp, y, z64):
    return _agg_kernel()(srcp, dstp, y, z64)


def _tc_a_body(x_ref, w1a_ref, w1b_ref, stem_ref, degp_ref, y_ref, dinv_ref):
    deg = degp_ref[0, :, 0:1] + degp_ref[1, :, 0:1] + 1.0
    dinv = lax.rsqrt(deg)
    xw = jnp.dot(x_ref[...], w1a_ref[...], preferred_element_type=jnp.float32)
    xw = xw + stem_ref[:, 0:1] * w1b_ref[...]
    y_ref[...] = xw * dinv
    dinv_ref[...] = dinv


def _tc_a(xp, w1a, w1b, stem, degp):
    return pl.pallas_call(
        _tc_a_body,
        grid=(GRID,),
        in_specs=[
            pl.BlockSpec((BR, D), lambda i: (i, 0)),
            pl.BlockSpec((D, H), lambda i: (0, 0)),
            pl.BlockSpec((1, H), lambda i: (0, 0)),
            pl.BlockSpec((BR, DW), lambda i: (i, 0)),
            pl.BlockSpec((NC, BR, DW), lambda i: (0, i, 0)),
        ],
        out_specs=[
            pl.BlockSpec((BR, H), lambda i: (i, 0)),
            pl.BlockSpec((BR, 1), lambda i: (i, 0)),
        ],
        out_shape=[jax.ShapeDtypeStruct((NPAD, H), jnp.float32),
                   jax.ShapeDtypeStruct((NPAD, 1), jnp.float32)],
    )(xp, w1a, w1b, stem, degp)


def _tc_b_body(y1_ref, agg_ref, dinv_ref, b1_ref, w2_ref, y2_ref):
    dinv = dinv_ref[...]
    h = dinv * (agg_ref[0, :, :] + agg_ref[1, :, :] + y1_ref[...]) + b1_ref[...]
    h = jnp.maximum(h, 0.0)
    y2_ref[...] = jnp.dot(h, w2_ref[...],
                          preferred_element_type=jnp.float32) * dinv


def _tc_b(y1, agg1, dinv, b1r, W2):
    return pl.pallas_call(
        _tc_b_body,
        grid=(GRID,),
        in_specs=[
            pl.BlockSpec((BR, H), lambda i: (i, 0)),
            pl.BlockSpec((NC, BR, H), lambda i: (0, i, 0)),
            pl.BlockSpec((BR, 1), lambda i: (i, 0)),
            pl.BlockSpec((1, H), lambda i: (0, 0)),
            pl.BlockSpec((H, H), lambda i: (0, 0)),
        ],
        out_specs=pl.BlockSpec((BR, H), lambda i: (i, 0)),
        out_shape=jax.ShapeDtypeStruct((NPAD, H), jnp.float32),
    )(y1, agg1, dinv, b1r, W2)


def _tc_c_body(y2_ref, agg_ref, dinv_ref, b2_ref, wb_ref, bb_ref, ws_ref,
               bs_ref, gb_ref, gs_ref, bl_ref, sl_ref, sb_ref, ss_ref):
    dinv = dinv_ref[...]
    h = dinv * (agg_ref[0, :, :] + agg_ref[1, :, :] + y2_ref[...]) + b2_ref[...]
    h = jnp.maximum(h, 0.0)
    bl = jnp.dot(h, wb_ref[...], preferred_element_type=jnp.float32) + bb_ref[...]
    sl = jnp.dot(h, ws_ref[...], preferred_element_type=jnp.float32) + bs_ref[...]
    bl_ref[...] = bl
    sl_ref[...] = sl
    for z, out in ((bl + gb_ref[...], sb_ref), (sl + gs_ref[...], ss_ref)):
        m = jnp.max(z, axis=1, keepdims=True)
        e = jnp.exp(z - m)
        out[...] = e / jnp.sum(e, axis=1, keepdims=True)


def _tc_c(y2, agg2, dinv, b2r, Wb, bbr, Ws, bsr, gb, gs):
    return pl.pallas_call(
        _tc_c_body,
        grid=(GRID,),
        in_specs=[
            pl.BlockSpec((BR, H), lambda i: (i, 0)),
            pl.BlockSpec((NC, BR, H), lambda i: (0, i, 0)),
            pl.BlockSpec((BR, 1), lambda i: (i, 0)),
            pl.BlockSpec((1, H), lambda i: (0, 0)),
            pl.BlockSpec((H, NB), lambda i: (0, 0)),
            pl.BlockSpec((1, NB), lambda i: (0, 0)),
            pl.BlockSpec((H, NS), lambda i: (0, 0)),
            pl.BlockSpec((1, NS), lambda i: (0, 0)),
            pl.BlockSpec((BR, NB), lambda i: (i, 0)),
            pl.BlockSpec((BR, NS), lambda i: (i, 0)),
        ],
        out_specs=[
            pl.BlockSpec((BR, NB), lambda i: (i, 0)),
            pl.BlockSpec((BR, NS), lambda i: (i, 0)),
            pl.BlockSpec((BR, NB), lambda i: (i, 0)),
            pl.BlockSpec((BR, NS), lambda i: (i, 0)),
        ],
        out_shape=[jax.ShapeDtypeStruct((N, NB), jnp.float32),
                   jax.ShapeDtypeStruct((N, NS), jnp.float32),
                   jax.ShapeDtypeStruct((N, NB), jnp.float32),
                   jax.ShapeDtypeStruct((N, NS), jnp.float32)],
    )(y2, agg2, dinv, b2r, Wb, bbr, Ws, bsr, gb, gs)


def kernel(x, edge_index, edge_attr, stem_idxs, W1, b1, W2, b2, Wb, bb, Ws, bs):
    f32 = jnp.float32
    i32 = jnp.int32
    src = edge_index[0].astype(i32)
    dst = edge_index[1].astype(i32)
    padv = jnp.full((E_PAD - E,), N, i32)
    srcp = jnp.concatenate([src, padv]).reshape(ALLOC_CH, CHUNK)
    dstp = jnp.concatenate([dst, padv]).reshape(ALLOC_CH, CHUNK)
    stemp = jnp.concatenate(
        [stem_idxs.astype(i32),
         jnp.full((4 * CHUNK - stem_idxs.shape[0],), N, i32)]).reshape(4, CHUNK)
    xp = jnp.pad(x.astype(f32), ((0, NPAD - N), (0, 0)))
    ones_col = jnp.ones((CHUNK, DW), f32)
    z1 = jnp.zeros((SLAB, DW), f32)
    z64 = jnp.zeros((SLAB, H), f32)

    degp, stem = _deg_stem(dstp, stemp, ones_col, z1)
    y1, dinv = _tc_a(xp, W1[:D], W1[D:D + 1], stem, degp)
    agg1 = _agg(srcp, dstp, y1, z64)
    y2 = _tc_b(y1, agg1, dinv, b1.reshape(1, H), W2)
    agg2 = _agg(srcp, dstp, y2, z64)
    gb, gs = _gumbels()
    bl, sl, sb, ss_o = _tc_c(y2, agg2, dinv, b2.reshape(1, H),
                             Wb, bb.reshape(1, NB), Ws, bs.reshape(1, NS),
                             jnp.asarray(gb), jnp.asarray(gs))
    return (bl, sl, sb, ss_o)


# speedup vs baseline: 2.6587x; 2.6587x over previous
import functools

import numpy as np
import jax
import jax.numpy as jnp
from jax import lax
from jax.experimental import pallas as pl
from jax.experimental.pallas import tpu as pltpu
from jax.experimental.pallas import tpu_sc as plsc

N = 10000
D = 128
H = 64
NB = 105
NS = 105
E = 320000

NC = 2
NSC = 16
NW = NC * NSC
NPAD = 10240
SLAB = NPAD // NSC
CHUNK = 128
CPT = 80
CPT0 = 80
CPT1 = 80
CH0 = NSC * CPT0
TOT_CH = NSC * (CPT0 + CPT1)
WCPT = max(CPT0, CPT1)
ALLOC_CH = max((NSC - 1) * CPT0, CH0 + (NSC - 1) * CPT1) + WCPT
E_PAD = ALLOC_CH * CHUNK
NBUF = 4
DW = 16

BR = 1024
GRID = NPAD // BR

@functools.lru_cache
def _mesh():
    return plsc.VectorSubcoreMesh(core_axis_name="c", subcore_axis_name="s",
                                  num_cores=NC, num_subcores=NSC)


_GCACHE = []


def _gumbels():
    if _GCACHE:
        return _GCACHE[0]

    def build():
        gk1, gk2 = jax.random.split(jax.random.key(42))
        outs = []
        for k, n in ((gk1, NB), (gk2, NS)):
            u = jax.random.uniform(k, (N, n), minval=1e-6, maxval=1.0 - 1e-6)
            outs.append(jnp.pad(-jnp.log(-jnp.log(u)), ((0, NPAD - N), (0, 0))))
        return outs

    try:
        with jax.ensure_compile_time_eval():
            gb, gs = build()
        gb, gs = np.asarray(gb), np.asarray(gs)
        _GCACHE.append((gb, gs))
        return gb, gs
    except Exception:
        return build()


def _deg_stem_body(dst_hbm, stem_hbm, ones_hbm, z1_hbm, degp_out, stem_out,
                   dstv, stemv, onesv, deg_sp, stem_sp, sem, sem2):
    cid = lax.axis_index("c")
    sid = lax.axis_index("s")
    wid = sid * NC + cid
    row0 = sid * SLAB
    pltpu.sync_copy(z1_hbm, deg_sp.at[pl.ds(row0, SLAB)])

    @pl.when(cid == 0)
    def _():
        pltpu.sync_copy(z1_hbm, stem_sp.at[pl.ds(row0, SLAB)])

    pltpu.sync_copy(ones_hbm, onesv)
    pltpu.sync_copy(dst_hbm.at[pl.ds(wid * CPT, CPT)], dstv)
    is00 = jnp.logical_and(cid == 0, sid == 0)

    @pl.when(is00)
    def _():
        pltpu.sync_copy(stem_hbm, stemv)

    plsc.subcore_barrier()

    def body(i, carry):
        for b in range(4):
            pltpu.async_copy(onesv, deg_sp.at[dstv.at[i * 4 + b]], sem, add=True)
        for _ in range(4):
            pltpu.make_async_copy(onesv, deg_sp.at[dstv.at[0]], sem).wait()
        return carry

    lax.fori_loop(0, CPT // 4, body, 0)

    @pl.when(is00)
    def _():
        for j in range(4):
            pltpu.async_copy(onesv, stem_sp.at[stemv.at[j]], sem2)
        for _ in range(4):
            pltpu.make_async_copy(onesv, stem_sp.at[stemv.at[0]], sem2).wait()

    plsc.subcore_barrier()
    pltpu.sync_copy(deg_sp.at[pl.ds(row0, SLAB)],
                    degp_out.at[cid, pl.ds(row0, SLAB)])

    @pl.when(cid == 0)
    def _():
        pltpu.sync_copy(stem_sp.at[pl.ds(row0, SLAB)],
                        stem_out.at[pl.ds(row0, SLAB)])


def _agg_body(src_hbm, dst_hbm, y_hbm, z64_hbm, out,
              srcv, dstv, agg_sp, buf0, buf1, buf2, buf3,
              g0, g1, g2, g3, s0, s1, s2, s3):
    bufs = (buf0, buf1, buf2, buf3)
    gs = (g0, g1, g2, g3)
    ss = (s0, s1, s2, s3)
    cid = lax.axis_index("c")
    sid = lax.axis_index("s")
    row0 = sid * SLAB
    base = jnp.where(cid == 0, sid * CPT0, CH0 + sid * CPT1)
    pltpu.sync_copy(z64_hbm, agg_sp.at[pl.ds(row0, SLAB)])
    pltpu.sync_copy(src_hbm.at[pl.ds(base, WCPT)], srcv)
    pltpu.sync_copy(dst_hbm.at[pl.ds(base, WCPT)], dstv)
    plsc.subcore_barrier()

    def run(my_cpt):
        if my_cpt == 0:
            return
        for b in range(NBUF):
            pltpu.async_copy(y_hbm.at[srcv.at[b]], bufs[b], gs[b])

        def body(i, carry):
            for b in range(NBUF):
                c = i * NBUF + b
                pltpu.make_async_copy(y_hbm.at[srcv.at[0]], bufs[b],
                                      gs[b]).wait()
                pltpu.async_copy(bufs[b], agg_sp.at[dstv.at[c]], ss[b],
                                 add=True)
                pltpu.make_async_copy(bufs[b], agg_sp.at[dstv.at[0]],
                                      ss[b]).wait()
                nc = c + NBUF

                @pl.when(nc < my_cpt)
                def _():
                    pltpu.async_copy(y_hbm.at[srcv.at[nc]], bufs[b], gs[b])
            return carry

        lax.fori_loop(0, my_cpt // NBUF, body, 0)

    @pl.when(cid == 0)
    def _():
        run(CPT0)

    @pl.when(cid == 1)
    def _():
        run(CPT1)

    plsc.subcore_barrier()
    pltpu.sync_copy(agg_sp.at[pl.ds(row0, SLAB)],
                    out.at[cid, pl.ds(row0, SLAB)])


@functools.lru_cache
def _deg_stem_kernel():
    return pl.kernel(
        _deg_stem_body,
        out_type=(jax.ShapeDtypeStruct((NC, NPAD, DW), jnp.float32),
                  jax.ShapeDtypeStruct((NPAD, DW), jnp.float32)),
        mesh=_mesh(),
        scratch_types=[
            pltpu.VMEM((CPT, CHUNK), jnp.int32),
            pltpu.VMEM((4, CHUNK), jnp.int32),
            pltpu.VMEM((CHUNK, DW), jnp.float32),
            pltpu.VMEM_SHARED((NPAD, DW), jnp.float32),
            pltpu.VMEM_SHARED((NPAD, DW), jnp.float32),
            pltpu.SemaphoreType.DMA,
            pltpu.SemaphoreType.DMA,
        ],
        compiler_params=pltpu.CompilerParams(use_tc_tiling_on_sc=False),
    )


def _deg_stem(dstp, stemp, ones_col, z1):
    return _deg_stem_kernel()(dstp, stemp, ones_col, z1)


@functools.lru_cache
def _agg_kernel():
    return pl.kernel(
        _agg_body,
        out_type=jax.ShapeDtypeStruct((NC, NPAD, H), jnp.float32),
        mesh=_mesh(),
        scratch_types=[
            pltpu.VMEM((WCPT, CHUNK), jnp.int32),
            pltpu.VMEM((WCPT, CHUNK), jnp.int32),
            pltpu.VMEM_SHARED((NPAD, H), jnp.float32),
            pltpu.VMEM((CHUNK, H), jnp.float32),
            pltpu.VMEM((CHUNK, H), jnp.float32),
            pltpu.VMEM((CHUNK, H), jnp.float32),
            pltpu.VMEM((CHUNK, H), jnp.float32),
            pltpu.SemaphoreType.DMA,
            pltpu.SemaphoreType.DMA,
            pltpu.SemaphoreType.DMA,
            pltpu.SemaphoreType.DMA,
            pltpu.SemaphoreType.DMA,
            pltpu.SemaphoreType.DMA,
            pltpu.SemaphoreType.DMA,
            pltpu.SemaphoreType.DMA,
        ],
        compiler_params=pltpu.CompilerParams(use_tc_tiling_on_sc=False),
    )


def _agg(srcp, dstp, y, z64):
    return _agg_kernel()(srcp, dstp, y, z64)


def _tc_a_body(x_ref, w1a_ref, w1b_ref, stem_ref, degp_ref, y_ref, dinv_ref):
    deg = degp_ref[0, :, 0:1] + degp_ref[1, :, 0:1] + 1.0
    dinv = lax.rsqrt(deg)
    xw = jnp.dot(x_ref[...], w1a_ref[...], preferred_element_type=jnp.float32)
    xw = xw + stem_ref[:, 0:1] * w1b_ref[...]
    y_ref[...] = xw * dinv
    dinv_ref[...] = dinv


def _tc_a(xp, w1a, w1b, stem, degp):
    return pl.pallas_call(
        _tc_a_body,
        grid=(GRID,),
        in_specs=[
            pl.BlockSpec((BR, D), lambda i: (i, 0)),
            pl.BlockSpec((D, H), lambda i: (0, 0)),
            pl.BlockSpec((1, H), lambda i: (0, 0)),
            pl.BlockSpec((BR, DW), lambda i: (i, 0)),
            pl.BlockSpec((NC, BR, DW), lambda i: (0, i, 0)),
        ],
        out_specs=[
            pl.BlockSpec((BR, H), lambda i: (i, 0)),
            pl.BlockSpec((BR, 1), lambda i: (i, 0)),
        ],
        out_shape=[jax.ShapeDtypeStruct((NPAD, H), jnp.float32),
                   jax.ShapeDtypeStruct((NPAD, 1), jnp.float32)],
    )(xp, w1a, w1b, stem, degp)


def _tc_b_body(y1_ref, agg_ref, dinv_ref, b1_ref, w2_ref, y2_ref):
    dinv = dinv_ref[...]
    h = dinv * (agg_ref[0, :, :] + agg_ref[1, :, :] + y1_ref[...]) + b1_ref[...]
    h = jnp.maximum(h, 0.0)
    y2_ref[...] = jnp.dot(h, w2_ref[...],
                          preferred_element_type=jnp.float32) * dinv


def _tc_b(y1, agg1, dinv, b1r, W2):
    return pl.pallas_call(
        _tc_b_body,
        grid=(GRID,),
        in_specs=[
            pl.BlockSpec((BR, H), lambda i: (i, 0)),
            pl.BlockSpec((NC, BR, H), lambda i: (0, i, 0)),
            pl.BlockSpec((BR, 1), lambda i: (i, 0)),
            pl.BlockSpec((1, H), lambda i: (0, 0)),
            pl.BlockSpec((H, H), lambda i: (0, 0)),
        ],
        out_specs=pl.BlockSpec((BR, H), lambda i: (i, 0)),
        out_shape=jax.ShapeDtypeStruct((NPAD, H), jnp.float32),
    )(y1, agg1, dinv, b1r, W2)


def _tc_c_body(y2_ref, agg_ref, dinv_ref, b2_ref, wb_ref, bb_ref, ws_ref,
               bs_ref, gb_ref, gs_ref, bl_ref, sl_ref, sb_ref, ss_ref):
    dinv = dinv_ref[...]
    h = dinv * (agg_ref[0, :, :] + agg_ref[1, :, :] + y2_ref[...]) + b2_ref[...]
    h = jnp.maximum(h, 0.0)
    bl = jnp.dot(h, wb_ref[...], preferred_element_type=jnp.float32) + bb_ref[...]
    sl = jnp.dot(h, ws_ref[...], preferred_element_type=jnp.float32) + bs_ref[...]
    bl_ref[...] = bl
    sl_ref[...] = sl
    for z, out in ((bl + gb_ref[...], sb_ref), (sl + gs_ref[...], ss_ref)):
        m = jnp.max(z, axis=1, keepdims=True)
        e = jnp.exp(z - m)
        out[...] = e / jnp.sum(e, axis=1, keepdims=True)


def _tc_c(y2, agg2, dinv, b2r, Wb, bbr, Ws, bsr, gb, gs):
    return pl.pallas_call(
        _tc_c_body,
        grid=(GRID,),
        in_specs=[
            pl.BlockSpec((BR, H), lambda i: (i, 0)),
            pl.BlockSpec((NC, BR, H), lambda i: (0, i, 0)),
            pl.BlockSpec((BR, 1), lambda i: (i, 0)),
            pl.BlockSpec((1, H), lambda i: (0, 0)),
            pl.BlockSpec((H, NB), lambda i: (0, 0)),
            pl.BlockSpec((1, NB), lambda i: (0, 0)),
            pl.BlockSpec((H, NS), lambda i: (0, 0)),
            pl.BlockSpec((1, NS), lambda i: (0, 0)),
            pl.BlockSpec((BR, NB), lambda i: (i, 0)),
            pl.BlockSpec((BR, NS), lambda i: (i, 0)),
        ],
        out_specs=[
            pl.BlockSpec((BR, NB), lambda i: (i, 0)),
            pl.BlockSpec((BR, NS), lambda i: (i, 0)),
            pl.BlockSpec((BR, NB), lambda i: (i, 0)),
            pl.BlockSpec((BR, NS), lambda i: (i, 0)),
        ],
        out_shape=[jax.ShapeDtypeStruct((N, NB), jnp.float32),
                   jax.ShapeDtypeStruct((N, NS), jnp.float32),
                   jax.ShapeDtypeStruct((N, NB), jnp.float32),
                   jax.ShapeDtypeStruct((N, NS), jnp.float32)],
    )(y2, agg2, dinv, b2r, Wb, bbr, Ws, bsr, gb, gs)


def kernel(x, edge_index, edge_attr, stem_idxs, W1, b1, W2, b2, Wb, bb, Ws, bs):
    f32 = jnp.float32
    i32 = jnp.int32
    src = edge_index[0].astype(i32)
    dst = edge_index[1].astype(i32)
    padv = N + jnp.arange(E_PAD - E, dtype=i32) % (NPAD - N)
    srcp = jnp.concatenate([src, padv]).reshape(ALLOC_CH, CHUNK)
    dstp = jnp.concatenate([dst, padv]).reshape(ALLOC_CH, CHUNK)
    stemp = jnp.concatenate(
        [stem_idxs.astype(i32),
         jnp.full((4 * CHUNK - stem_idxs.shape[0],), N, i32)]).reshape(4, CHUNK)
    xp = jnp.pad(x.astype(f32), ((0, NPAD - N), (0, 0)))
    ones_col = jnp.ones((CHUNK, DW), f32)
    z1 = jnp.zeros((SLAB, DW), f32)
    z64 = jnp.zeros((SLAB, H), f32)

    degp, stem = _deg_stem(dstp, stemp, ones_col, z1)
    y1, dinv = _tc_a(xp, W1[:D], W1[D:D + 1], stem, degp)
    agg1 = _agg(srcp, dstp, y1, z64)
    y2 = _tc_b(y1, agg1, dinv, b1.reshape(1, H), W2)
    agg2 = _agg(srcp, dstp, y2, z64)
    gb, gs = _gumbels()
    bl, sl, sb, ss_o = _tc_c(y2, agg2, dinv, b2.reshape(1, H),
                             Wb, bb.reshape(1, NB), Ws, bs.reshape(1, NS),
                             jnp.asarray(gb), jnp.asarray(gs))
    return (bl, sl, sb, ss_o)


# speedup vs baseline: 2.6645x; 1.0022x over previous
import functools

import numpy as np
import jax
import jax.numpy as jnp
from jax import lax
from jax.experimental import pallas as pl
from jax.experimental.pallas import tpu as pltpu
from jax.experimental.pallas import tpu_sc as plsc

N = 10000
D = 128
H = 64
NB = 105
NS = 105
E = 320000

NC = 2
NSC = 16
NW = NC * NSC
NPAD = 10240
SLAB = NPAD // NSC
CHUNK = 128
CPT = 80
CPT0 = 80
CPT1 = 80
CH0 = NSC * CPT0
TOT_CH = NSC * (CPT0 + CPT1)
WCPT = max(CPT0, CPT1)
ALLOC_CH = max((NSC - 1) * CPT0, CH0 + (NSC - 1) * CPT1) + WCPT
E_PAD = ALLOC_CH * CHUNK
NBUF = 4
DW = 16

BR = 1024
GRID = NPAD // BR

@functools.lru_cache
def _mesh():
    return plsc.VectorSubcoreMesh(core_axis_name="c", subcore_axis_name="s",
                                  num_cores=NC, num_subcores=NSC)


_GCACHE = []


def _gumbels():
    if _GCACHE:
        return _GCACHE[0]

    def build():
        gk1, gk2 = jax.random.split(jax.random.key(42))
        outs = []
        for k, n in ((gk1, NB), (gk2, NS)):
            u = jax.random.uniform(k, (N, n), minval=1e-6, maxval=1.0 - 1e-6)
            outs.append(jnp.pad(-jnp.log(-jnp.log(u)), ((0, NPAD - N), (0, 0))))
        return outs

    try:
        with jax.ensure_compile_time_eval():
            gb, gs = build()
        gb, gs = np.asarray(gb), np.asarray(gs)
        _GCACHE.append((gb, gs))
        return gb, gs
    except Exception:
        return build()


def _deg_stem_body(dst_hbm, stem_hbm, ones_hbm, z1_hbm, degp_out, stem_out,
                   dstv, stemv, onesv, deg_sp, stem_sp, sem, sem2):
    cid = lax.axis_index("c")
    sid = lax.axis_index("s")
    wid = sid * NC + cid
    row0 = sid * SLAB
    pltpu.sync_copy(z1_hbm, deg_sp.at[pl.ds(row0, SLAB)])

    @pl.when(cid == 0)
    def _():
        pltpu.sync_copy(z1_hbm, stem_sp.at[pl.ds(row0, SLAB)])

    pltpu.sync_copy(ones_hbm, onesv)
    pltpu.sync_copy(dst_hbm.at[pl.ds(wid * CPT, CPT)], dstv)
    is00 = jnp.logical_and(cid == 0, sid == 0)

    @pl.when(is00)
    def _():
        pltpu.sync_copy(stem_hbm, stemv)

    plsc.subcore_barrier()

    def body(i, carry):
        for b in range(4):
            pltpu.async_copy(onesv, deg_sp.at[dstv.at[i * 4 + b]], sem, add=True)
        for b in range(4):
            pltpu.make_async_copy(onesv, deg_sp.at[dstv.at[i * 4 + b]], sem).wait()
        return carry

    lax.fori_loop(0, CPT // 4, body, 0)

    @pl.when(is00)
    def _():
        for j in range(4):
            pltpu.async_copy(onesv, stem_sp.at[stemv.at[j]], sem2)
        for j in range(4):
            pltpu.make_async_copy(onesv, stem_sp.at[stemv.at[j]], sem2).wait()

    plsc.subcore_barrier()
    pltpu.sync_copy(deg_sp.at[pl.ds(row0, SLAB)],
                    degp_out.at[cid, pl.ds(row0, SLAB)])

    @pl.when(cid == 0)
    def _():
        pltpu.sync_copy(stem_sp.at[pl.ds(row0, SLAB)],
                        stem_out.at[pl.ds(row0, SLAB)])


def _agg_body(src_hbm, dst_hbm, y_hbm, z64_hbm, out,
              srcv, dstv, agg_sp, buf0, buf1, buf2, buf3,
              g0, g1, g2, g3, s0, s1, s2, s3):
    bufs = (buf0, buf1, buf2, buf3)
    gs = (g0, g1, g2, g3)
    ss = (s0, s1, s2, s3)
    cid = lax.axis_index("c")
    sid = lax.axis_index("s")
    row0 = sid * SLAB
    base = jnp.where(cid == 0, sid * CPT0, CH0 + sid * CPT1)
    pltpu.sync_copy(z64_hbm, agg_sp.at[pl.ds(row0, SLAB)])
    pltpu.sync_copy(src_hbm.at[pl.ds(base, WCPT)], srcv)
    pltpu.sync_copy(dst_hbm.at[pl.ds(base, WCPT)], dstv)
    plsc.subcore_barrier()

    def run(my_cpt):
        if my_cpt == 0:
            return
        for b in range(NBUF):
            pltpu.async_copy(y_hbm.at[srcv.at[b]], bufs[b], gs[b])

        def body(i, carry):
            for b in range(NBUF):
                c = i * NBUF + b
                pltpu.make_async_copy(y_hbm.at[srcv.at[c]], bufs[b],
                                      gs[b]).wait()
                pltpu.async_copy(bufs[b], agg_sp.at[dstv.at[c]], ss[b],
                                 add=True)
                pltpu.make_async_copy(bufs[b], agg_sp.at[dstv.at[c]],
                                      ss[b]).wait()
                nc = c + NBUF

                @pl.when(nc < my_cpt)
                def _():
                    pltpu.async_copy(y_hbm.at[srcv.at[nc]], bufs[b], gs[b])
            return carry

        lax.fori_loop(0, my_cpt // NBUF, body, 0)

    @pl.when(cid == 0)
    def _():
        run(CPT0)

    @pl.when(cid == 1)
    def _():
        run(CPT1)

    plsc.subcore_barrier()
    pltpu.sync_copy(agg_sp.at[pl.ds(row0, SLAB)],
                    out.at[cid, pl.ds(row0, SLAB)])


@functools.lru_cache
def _deg_stem_kernel():
    return pl.kernel(
        _deg_stem_body,
        out_type=(jax.ShapeDtypeStruct((NC, NPAD, DW), jnp.float32),
                  jax.ShapeDtypeStruct((NPAD, DW), jnp.float32)),
        mesh=_mesh(),
        scratch_types=[
            pltpu.VMEM((CPT, CHUNK), jnp.int32),
            pltpu.VMEM((4, CHUNK), jnp.int32),
            pltpu.VMEM((CHUNK, DW), jnp.float32),
            pltpu.VMEM_SHARED((NPAD, DW), jnp.float32),
            pltpu.VMEM_SHARED((NPAD, DW), jnp.float32),
            pltpu.SemaphoreType.DMA,
            pltpu.SemaphoreType.DMA,
        ],
        compiler_params=pltpu.CompilerParams(use_tc_tiling_on_sc=False),
    )


def _deg_stem(dstp, stemp, ones_col, z1):
    return _deg_stem_kernel()(dstp, stemp, ones_col, z1)


@functools.lru_cache
def _agg_kernel():
    return pl.kernel(
        _agg_body,
        out_type=jax.ShapeDtypeStruct((NC, NPAD, H), jnp.float32),
        mesh=_mesh(),
        scratch_types=[
            pltpu.VMEM((WCPT, CHUNK), jnp.int32),
            pltpu.VMEM((WCPT, CHUNK), jnp.int32),
            pltpu.VMEM_SHARED((NPAD, H), jnp.float32),
            pltpu.VMEM((CHUNK, H), jnp.float32),
            pltpu.VMEM((CHUNK, H), jnp.float32),
            pltpu.VMEM((CHUNK, H), jnp.float32),
            pltpu.VMEM((CHUNK, H), jnp.float32),
            pltpu.SemaphoreType.DMA,
            pltpu.SemaphoreType.DMA,
            pltpu.SemaphoreType.DMA,
            pltpu.SemaphoreType.DMA,
            pltpu.SemaphoreType.DMA,
            pltpu.SemaphoreType.DMA,
            pltpu.SemaphoreType.DMA,
            pltpu.SemaphoreType.DMA,
        ],
        compiler_params=pltpu.CompilerParams(use_tc_tiling_on_sc=False),
    )


def _agg(srcp, dstp, y, z64):
    return _agg_kernel()(srcp, dstp, y, z64)


def _tc_a_body(x_ref, w1a_ref, w1b_ref, stem_ref, degp_ref, y_ref, dinv_ref):
    deg = degp_ref[0, :, 0:1] + degp_ref[1, :, 0:1] + 1.0
    dinv = lax.rsqrt(deg)
    xw = jnp.dot(x_ref[...], w1a_ref[...], preferred_element_type=jnp.float32)
    xw = xw + stem_ref[:, 0:1] * w1b_ref[...]
    y_ref[...] = xw * dinv
    dinv_ref[...] = dinv


def _tc_a(xp, w1a, w1b, stem, degp):
    return pl.pallas_call(
        _tc_a_body,
        grid=(GRID,),
        in_specs=[
            pl.BlockSpec((BR, D), lambda i: (i, 0)),
            pl.BlockSpec((D, H), lambda i: (0, 0)),
            pl.BlockSpec((1, H), lambda i: (0, 0)),
            pl.BlockSpec((BR, DW), lambda i: (i, 0)),
            pl.BlockSpec((NC, BR, DW), lambda i: (0, i, 0)),
        ],
        out_specs=[
            pl.BlockSpec((BR, H), lambda i: (i, 0)),
            pl.BlockSpec((BR, 1), lambda i: (i, 0)),
        ],
        out_shape=[jax.ShapeDtypeStruct((NPAD, H), jnp.float32),
                   jax.ShapeDtypeStruct((NPAD, 1), jnp.float32)],
    )(xp, w1a, w1b, stem, degp)


def _tc_b_body(y1_ref, agg_ref, dinv_ref, b1_ref, w2_ref, y2_ref):
    dinv = dinv_ref[...]
    h = dinv * (agg_ref[0, :, :] + agg_ref[1, :, :] + y1_ref[...]) + b1_ref[...]
    h = jnp.maximum(h, 0.0)
    y2_ref[...] = jnp.dot(h, w2_ref[...],
                          preferred_element_type=jnp.float32) * dinv


def _tc_b(y1, agg1, dinv, b1r, W2):
    return pl.pallas_call(
        _tc_b_body,
        grid=(GRID,),
        in_specs=[
            pl.BlockSpec((BR, H), lambda i: (i, 0)),
            pl.BlockSpec((NC, BR, H), lambda i: (0, i, 0)),
            pl.BlockSpec((BR, 1), lambda i: (i, 0)),
            pl.BlockSpec((1, H), lambda i: (0, 0)),
            pl.BlockSpec((H, H), lambda i: (0, 0)),
        ],
        out_specs=pl.BlockSpec((BR, H), lambda i: (i, 0)),
        out_shape=jax.ShapeDtypeStruct((NPAD, H), jnp.float32),
    )(y1, agg1, dinv, b1r, W2)


def _tc_c_body(y2_ref, agg_ref, dinv_ref, b2_ref, wb_ref, bb_ref, ws_ref,
               bs_ref, gb_ref, gs_ref, bl_ref, sl_ref, sb_ref, ss_ref):
    dinv = dinv_ref[...]
    h = dinv * (agg_ref[0, :, :] + agg_ref[1, :, :] + y2_ref[...]) + b2_ref[...]
    h = jnp.maximum(h, 0.0)
    bl = jnp.dot(h, wb_ref[...], preferred_element_type=jnp.float32) + bb_ref[...]
    sl = jnp.dot(h, ws_ref[...], preferred_element_type=jnp.float32) + bs_ref[...]
    bl_ref[...] = bl
    sl_ref[...] = sl
    for z, out in ((bl + gb_ref[...], sb_ref), (sl + gs_ref[...], ss_ref)):
        m = jnp.max(z, axis=1, keepdims=True)
        e = jnp.exp(z - m)
        out[...] = e / jnp.sum(e, axis=1, keepdims=True)


def _tc_c(y2, agg2, dinv, b2r, Wb, bbr, Ws, bsr, gb, gs):
    return pl.pallas_call(
        _tc_c_body,
        grid=(GRID,),
        in_specs=[
            pl.BlockSpec((BR, H), lambda i: (i, 0)),
            pl.BlockSpec((NC, BR, H), lambda i: (0, i, 0)),
            pl.BlockSpec((BR, 1), lambda i: (i, 0)),
            pl.BlockSpec((1, H), lambda i: (0, 0)),
            pl.BlockSpec((H, NB), lambda i: (0, 0)),
            pl.BlockSpec((1, NB), lambda i: (0, 0)),
            pl.BlockSpec((H, NS), lambda i: (0, 0)),
            pl.BlockSpec((1, NS), lambda i: (0, 0)),
            pl.BlockSpec((BR, NB), lambda i: (i, 0)),
            pl.BlockSpec((BR, NS), lambda i: (i, 0)),
        ],
        out_specs=[
            pl.BlockSpec((BR, NB), lambda i: (i, 0)),
            pl.BlockSpec((BR, NS), lambda i: (i, 0)),
            pl.BlockSpec((BR, NB), lambda i: (i, 0)),
            pl.BlockSpec((BR, NS), lambda i: (i, 0)),
        ],
        out_shape=[jax.ShapeDtypeStruct((N, NB), jnp.float32),
                   jax.ShapeDtypeStruct((N, NS), jnp.float32),
                   jax.ShapeDtypeStruct((N, NB), jnp.float32),
                   jax.ShapeDtypeStruct((N, NS), jnp.float32)],
    )(y2, agg2, dinv, b2r, Wb, bbr, Ws, bsr, gb, gs)


def kernel(x, edge_index, edge_attr, stem_idxs, W1, b1, W2, b2, Wb, bb, Ws, bs):
    f32 = jnp.float32
    i32 = jnp.int32
    src = edge_index[0].astype(i32)
    dst = edge_index[1].astype(i32)
    padv = N + jnp.arange(E_PAD - E, dtype=i32) % (NPAD - N)
    srcp = jnp.concatenate([src, padv]).reshape(ALLOC_CH, CHUNK)
    dstp = jnp.concatenate([dst, padv]).reshape(ALLOC_CH, CHUNK)
    stemp = jnp.concatenate(
        [stem_idxs.astype(i32),
         jnp.full((4 * CHUNK - stem_idxs.shape[0],), N, i32)]).reshape(4, CHUNK)
    xp = jnp.pad(x.astype(f32), ((0, NPAD - N), (0, 0)))
    ones_col = jnp.ones((CHUNK, DW), f32)
    z1 = jnp.zeros((SLAB, DW), f32)
    z64 = jnp.zeros((SLAB, H), f32)

    degp, stem = _deg_stem(dstp, stemp, ones_col, z1)
    y1, dinv = _tc_a(xp, W1[:D], W1[D:D + 1], stem, degp)
    agg1 = _agg(srcp, dstp, y1, z64)
    y2 = _tc_b(y1, agg1, dinv, b1.reshape(1, H), W2)
    agg2 = _agg(srcp, dstp, y2, z64)
    gb, gs = _gumbels()
    bl, sl, sb, ss_o = _tc_c(y2, agg2, dinv, b2.reshape(1, H),
                             Wb, bb.reshape(1, NB), Ws, bs.reshape(1, NS),
                             jnp.asarray(gb), jnp.asarray(gs))
    return (bl, sl, sb, ss_o)
